# baseline (speedup 1.0000x reference)
import functools

import jax
import jax.numpy as jnp
import numpy as np
from jax import lax
from jax.experimental import pallas as pl
from jax.experimental.pallas import tpu as pltpu
from jax.experimental.pallas import tpu_sc as plsc

N = 10000
NP = 10240
E = 320000
P = 100000
D = 64
NC, NS = 2, 16
NW = NC * NS
CH = 128
EC = 80
EP = NW * EC * CH
PC = 26
PP = NW * PC * CH
RPT = NP // NS


def _mm(a, b):
    return lax.dot_general(a, b, (((1,), (1,)), ((), ())),
                           preferred_element_type=jnp.float32)


def _stage_a(x_ref, wl_ref, wr_ref, y1_ref, r1_ref):
    xb = x_ref[...]
    y1_ref[...] = _mm(xb, wl_ref[...])
    r1_ref[...] = _mm(xb, wr_ref[...])


def _stage_c(acc_ref, deg_ref, r1_ref, b1_ref, w2l_ref, w2r_ref, b2_ref,
             y2_ref, p2_ref, rec_ref):
    deg = deg_ref[0, :, 0:1] + deg_ref[1, :, 0:1]
    rec = 1.0 / jnp.maximum(deg, 1.0)
    agg = (acc_ref[0] + acc_ref[1]) * rec
    h1 = jnp.maximum(agg + b1_ref[...] + r1_ref[...], 0.0)
    y2_ref[...] = _mm(h1, w2l_ref[...])
    p2_ref[...] = _mm(h1, w2r_ref[...]) + b2_ref[...]
    rec_ref[...] = jnp.broadcast_to(rec, rec_ref.shape)


def _stage_e(acc_ref, rec_ref, p2_ref, wpa_ref, wpb_ref, u_ref, v_ref):
    h2 = (acc_ref[0] + acc_ref[1]) * rec_ref[...] + p2_ref[...]
    u_ref[...] = _mm(h2, wpa_ref[...])
    v_ref[...] = _mm(h2, wpb_ref[...])


def _stage_g(gu_ref, gv_ref, bp1c_ref, wp2c_ref, bp2_ref, p0_ref, p1_ref):
    z = jnp.maximum(gu_ref[...] + gv_ref[...] + bp1c_ref[...], 0.0)
    part = z * wp2c_ref[...]
    l0 = jnp.sum(part[:, :D], axis=1) + bp2_ref[0, 0]
    l1 = jnp.sum(part[:, D:], axis=1) + bp2_ref[0, 0]
    p0_ref[...] = jax.nn.sigmoid(l0)
    p1_ref[...] = jax.nn.sigmoid(l1)


def _seg_sum_body(with_deg, y_hbm, srcv_hbm, dstv_hbm, z64_hbm, z16_hbm,
                  ones_hbm, acc_hbm, deg_hbm, idxs_v, idxd_v,
                  rb0, rb1, rb2, rb3, zb, ones_v, z16_v, acc_s, deg_s,
                  gs0, gs1, gs2, gs3, ss0, ss1, ss2, ss3):
    c = lax.axis_index("c")
    s = lax.axis_index("s")
    wid = c * NS + s
    rbs = [rb0, rb1, rb2, rb3]
    gss = [gs0, gs1, gs2, gs3]
    sss = [ss0, ss1, ss2, ss3]
    pltpu.sync_copy(z64_hbm, zb)
    if with_deg:
        pltpu.sync_copy(z16_hbm, z16_v)
        pltpu.sync_copy(ones_hbm, ones_v)

    def zbody(k, carry):
        pltpu.sync_copy(zb, acc_s.at[pl.ds(s * RPT + k * CH, CH)])
        if with_deg:
            pltpu.sync_copy(z16_v, deg_s.at[pl.ds(s * RPT + k * CH, CH)])
        return carry

    lax.fori_loop(0, RPT // CH, zbody, 0)
    pltpu.sync_copy(srcv_hbm.at[wid], idxs_v)
    pltpu.sync_copy(dstv_hbm.at[wid], idxd_v)
    plsc.subcore_barrier()

    for k in range(4):
        pltpu.async_copy(y_hbm.at[idxs_v.at[k]], rbs[k], gss[k])

    def body(t, carry):
        for k in range(4):
            j = 4 * t + k
            pltpu.make_async_copy(y_hbm.at[idxs_v.at[0]], rbs[k],
                                  gss[k]).wait()
            pltpu.async_copy(rbs[k], acc_s.at[idxd_v.at[j]], sss[k],
                             add=True)
            if with_deg:
                pltpu.sync_copy(ones_v, deg_s.at[idxd_v.at[j]], add=True)
        for k in range(4):
            j = 4 * t + k
            pltpu.make_async_copy(rbs[k], acc_s.at[idxd_v.at[0]],
                                  sss[k]).wait()
            pltpu.async_copy(y_hbm.at[idxs_v.at[(j + 4) % EC]], rbs[k],
                             gss[k])
        return carry

    lax.fori_loop(0, EC // 4, body, 0)
    for k in range(4):
        pltpu.make_async_copy(y_hbm.at[idxs_v.at[0]], rbs[k], gss[k]).wait()
    plsc.subcore_barrier()

    def wbody(k, carry):
        base = s * RPT + k * CH
        pltpu.sync_copy(acc_s.at[pl.ds(base, CH)], rb0)
        pltpu.sync_copy(rb0, acc_hbm.at[c, pl.ds(base, CH)])
        if with_deg:
            pltpu.sync_copy(deg_s.at[pl.ds(base, CH)], z16_v)
            pltpu.sync_copy(z16_v, deg_hbm.at[c, pl.ds(base, CH)])
        return carry

    lax.fori_loop(0, RPT // CH, wbody, 0)


def _gather_pairs_body(u_hbm, v_hbm, sidx_hbm, didx_hbm, gu_hbm, gv_hbm,
                       idxs_v, idxd_v, bu_a, bv_a, bu_b, bv_b,
                       su_a, sv_a, su_b, sv_b):
    c = lax.axis_index("c")
    s = lax.axis_index("s")
    wid = c * NS + s
    pltpu.sync_copy(sidx_hbm.at[wid], idxs_v)
    pltpu.sync_copy(didx_hbm.at[wid], idxd_v)
    pltpu.async_copy(u_hbm.at[idxs_v.at[0]], bu_a, su_a)
    pltpu.async_copy(v_hbm.at[idxd_v.at[0]], bv_a, sv_a)

    def body(t, carry):
        j0 = 2 * t
        j1 = j0 + 1
        j2 = (j0 + 2) % PC
        pltpu.async_copy(u_hbm.at[idxs_v.at[j1]], bu_b, su_b)
        pltpu.async_copy(v_hbm.at[idxd_v.at[j1]], bv_b, sv_b)
        pltpu.make_async_copy(u_hbm.at[idxs_v.at[j0]], bu_a, su_a).wait()
        pltpu.make_async_copy(v_hbm.at[idxd_v.at[j0]], bv_a, sv_a).wait()
        base0 = (wid * PC + j0) * CH
        pltpu.sync_copy(bu_a, gu_hbm.at[pl.ds(base0, CH)])
        pltpu.sync_copy(bv_a, gv_hbm.at[pl.ds(base0, CH)])
        pltpu.async_copy(u_hbm.at[idxs_v.at[j2]], bu_a, su_a)
        pltpu.async_copy(v_hbm.at[idxd_v.at[j2]], bv_a, sv_a)
        pltpu.make_async_copy(u_hbm.at[idxs_v.at[j1]], bu_b, su_b).wait()
        pltpu.make_async_copy(v_hbm.at[idxd_v.at[j1]], bv_b, sv_b).wait()
        base1 = (wid * PC + j1) * CH
        pltpu.sync_copy(bu_b, gu_hbm.at[pl.ds(base1, CH)])
        pltpu.sync_copy(bv_b, gv_hbm.at[pl.ds(base1, CH)])
        return carry

    lax.fori_loop(0, PC // 2, body, 0)
    pltpu.make_async_copy(u_hbm.at[idxs_v.at[0]], bu_a, su_a).wait()
    pltpu.make_async_copy(v_hbm.at[idxd_v.at[0]], bv_a, sv_a).wait()


_SC_MESH = plsc.VectorSubcoreMesh(core_axis_name="c", subcore_axis_name="s",
                                  num_cores=NC, num_subcores=NS)

_SEG_SCRATCH = ([pltpu.VMEM((EC, CH), jnp.int32),
                 pltpu.VMEM((EC, CH), jnp.int32)]
                + [pltpu.VMEM((CH, D), jnp.float32)] * 5
                + [pltpu.VMEM((CH, 16), jnp.float32),
                   pltpu.VMEM((CH, 16), jnp.float32),
                   pltpu.VMEM_SHARED((NP, D), jnp.float32),
                   pltpu.VMEM_SHARED((NP, 16), jnp.float32)]
                + [pltpu.SemaphoreType.DMA] * 8)

_seg_sum_deg = functools.partial(
    pl.kernel, functools.partial(_seg_sum_body, True),
    out_type=(jax.ShapeDtypeStruct((NC, NP, D), jnp.float32),
              jax.ShapeDtypeStruct((NC, NP, 16), jnp.float32)),
    mesh=_SC_MESH,
    compiler_params=pltpu.CompilerParams(use_tc_tiling_on_sc=False),
    scratch_types=_SEG_SCRATCH,
)()

_seg_sum_nodeg = functools.partial(
    pl.kernel, functools.partial(_seg_sum_body, False),
    out_type=(jax.ShapeDtypeStruct((NC, NP, D), jnp.float32),
              jax.ShapeDtypeStruct((NC, NP, 16), jnp.float32)),
    mesh=_SC_MESH,
    compiler_params=pltpu.CompilerParams(use_tc_tiling_on_sc=False),
    scratch_types=_SEG_SCRATCH,
)()

_gather_pairs = functools.partial(
    pl.kernel, _gather_pairs_body,
    out_type=(jax.ShapeDtypeStruct((PP, D), jnp.float32),
              jax.ShapeDtypeStruct((PP, D), jnp.float32)),
    mesh=_SC_MESH,
    compiler_params=pltpu.CompilerParams(use_tc_tiling_on_sc=False),
    scratch_types=[pltpu.VMEM((PC, CH), jnp.int32),
                   pltpu.VMEM((PC, CH), jnp.int32),
                   pltpu.VMEM((CH, D), jnp.float32),
                   pltpu.VMEM((CH, D), jnp.float32),
                   pltpu.VMEM((CH, D), jnp.float32),
                   pltpu.VMEM((CH, D), jnp.float32),
                   pltpu.SemaphoreType.DMA,
                   pltpu.SemaphoreType.DMA,
                   pltpu.SemaphoreType.DMA,
                   pltpu.SemaphoreType.DMA],
)()


def kernel(x, edge_index, edge_label_index, W1_l, b1_l, W1_r,
           W2_l, b2_l, W2_r, Wp1, bp1, Wp2, bp2):
    f32 = jnp.float32
    ei = edge_index.astype(jnp.int32)
    epad_s = jnp.asarray(np.arange(EP - E, dtype=np.int32) % N)
    epad_d = jnp.asarray(N + np.arange(EP - E, dtype=np.int32) % (NP - N))
    srcv = jnp.concatenate([ei[0], epad_s]).reshape(NW, EC, CH)
    dstv = jnp.concatenate([ei[1], epad_d]).reshape(NW, EC, CH)
    eli = edge_label_index.astype(jnp.int32)
    ppad = jnp.asarray(np.arange(PP - P, dtype=np.int32) % N)
    sidx = jnp.concatenate([eli[0], ppad]).reshape(NW, PC, CH)
    didx = jnp.concatenate([eli[1], ppad]).reshape(NW, PC, CH)
    z64 = jnp.asarray(np.zeros((CH, D), np.float32))
    z16 = jnp.asarray(np.zeros((CH, 16), np.float32))
    ones16 = jnp.asarray(np.ones((CH, 16), np.float32))

    nb = pl.cdiv(N, 1024)
    row_spec = pl.BlockSpec((1024, D), lambda i: (i, 0))
    acc_spec = pl.BlockSpec((NC, 1024, D), lambda i: (0, i, 0))
    deg_spec = pl.BlockSpec((NC, 1024, 16), lambda i: (0, i, 0))
    w64_spec = pl.BlockSpec((D, D), lambda i: (0, 0))
    b_spec = pl.BlockSpec((1, D), lambda i: (0, 0))
    row_out = jax.ShapeDtypeStruct((N, D), f32)

    y1, r1 = pl.pallas_call(
        _stage_a,
        grid=(nb,),
        in_specs=[pl.BlockSpec((1024, 128), lambda i: (i, 0)),
                  pl.BlockSpec((D, 128), lambda i: (0, 0)),
                  pl.BlockSpec((D, 128), lambda i: (0, 0))],
        out_specs=[row_spec, row_spec],
        out_shape=[row_out, row_out],
    )(x, W1_l, W1_r)

    acc1, deg = _seg_sum_deg(y1, srcv, dstv, z64, z16, ones16)

    y2, p2, rec = pl.pallas_call(
        _stage_c,
        grid=(nb,),
        in_specs=[acc_spec, deg_spec, row_spec, b_spec, w64_spec, w64_spec,
                  b_spec],
        out_specs=[row_spec, row_spec, row_spec],
        out_shape=[row_out, row_out, row_out],
    )(acc1, deg, r1, b1_l.reshape(1, D), W2_l, W2_r, b2_l.reshape(1, D))

    acc2, _ = _seg_sum_nodeg(y2, srcv, dstv, z64, z16, ones16)

    u, v = pl.pallas_call(
        _stage_e,
        grid=(nb,),
        in_specs=[acc_spec, row_spec, row_spec, w64_spec, w64_spec],
        out_specs=[row_spec, row_spec],
        out_shape=[row_out, row_out],
    )(acc2, rec, p2, Wp1[:, :D], Wp1[:, D:])

    gu, gv = _gather_pairs(u, v, sidx, didx)

    PPH = PP // 2
    gu2 = gu.reshape(PPH, 2 * D)
    gv2 = gv.reshape(PPH, 2 * D)
    bp1c = jnp.concatenate([bp1, bp1]).reshape(1, 2 * D)
    wp2c = jnp.concatenate([Wp2[0], Wp2[0]]).reshape(1, 2 * D)
    GB = 4096
    pb = pl.cdiv(PPH, GB)
    p0, p1 = pl.pallas_call(
        _stage_g,
        grid=(pb,),
        in_specs=[pl.BlockSpec((GB, 2 * D), lambda i: (i, 0)),
                  pl.BlockSpec((GB, 2 * D), lambda i: (i, 0)),
                  pl.BlockSpec((1, 2 * D), lambda i: (0, 0)),
                  pl.BlockSpec((1, 2 * D), lambda i: (0, 0)),
                  pl.BlockSpec((1, 1), lambda i: (0, 0))],
        out_specs=[pl.BlockSpec((GB,), lambda i: (i,)),
                   pl.BlockSpec((GB,), lambda i: (i,))],
        out_shape=[jax.ShapeDtypeStruct((PPH,), f32),
                   jax.ShapeDtypeStruct((PPH,), f32)],
    )(gu2, gv2, bp1c, wp2c, bp2.reshape(1, 1))

    probs = jnp.stack([p0, p1], axis=-1).reshape(PP)
    return probs[:P]


# speedup vs baseline: 1.0666x; 1.0666x over previous
import functools

import jax
import jax.numpy as jnp
import numpy as np
from jax import lax
from jax.experimental import pallas as pl
from jax.experimental.pallas import tpu as pltpu
from jax.experimental.pallas import tpu_sc as plsc

N = 10000
NP = 10240
E = 320000
P = 100000
D = 64
NC, NS = 2, 16
NW = NC * NS
CH = 128
EC = 80
EP = NW * EC * CH
PC = 26
PP = NW * PC * CH
RPT = NP // NS


def _mm(a, b):
    return lax.dot_general(a, b, (((1,), (1,)), ((), ())),
                           preferred_element_type=jnp.float32)


def _stage_a(x_ref, wl_ref, wr_ref, y1_ref, r1_ref):
    xb = x_ref[...]
    y1_ref[...] = _mm(xb, wl_ref[...])
    r1_ref[...] = _mm(xb, wr_ref[...])


def _stage_c(acc_ref, deg_ref, r1_ref, b1_ref, w2l_ref, w2r_ref, b2_ref,
             y2_ref, p2_ref, rec_ref):
    deg = deg_ref[0, :, 0:1] + deg_ref[1, :, 0:1]
    rec = 1.0 / jnp.maximum(deg, 1.0)
    agg = (acc_ref[0] + acc_ref[1]) * rec
    h1 = jnp.maximum(agg + b1_ref[...] + r1_ref[...], 0.0)
    y2_ref[...] = _mm(h1, w2l_ref[...])
    p2_ref[...] = _mm(h1, w2r_ref[...]) + b2_ref[...]
    rec_ref[...] = jnp.broadcast_to(rec, rec_ref.shape)


def _stage_e(acc_ref, rec_ref, p2_ref, wpa_ref, wpb_ref, u_ref, v_ref):
    h2 = (acc_ref[0] + acc_ref[1]) * rec_ref[...] + p2_ref[...]
    u_ref[...] = _mm(h2, wpa_ref[...])
    v_ref[...] = _mm(h2, wpb_ref[...])


def _stage_g(gu_ref, gv_ref, bp1c_ref, wp2c_ref, bp2_ref, pr_ref):
    z = jnp.maximum(gu_ref[...] + gv_ref[...] + bp1c_ref[...], 0.0)
    part = z * wp2c_ref[...]
    l0 = jnp.sum(part[:, :D], axis=1) + bp2_ref[0, 0]
    l1 = jnp.sum(part[:, D:], axis=1) + bp2_ref[0, 0]
    pr_ref[...] = jax.nn.sigmoid(jnp.stack([l0, l1], axis=-1))


def _seg_sum_body(with_deg, y_hbm, srcv_hbm, dstv_hbm, z64_hbm, z16_hbm,
                  ones_hbm, acc_hbm, deg_hbm, idxs_v, idxd_v,
                  rb0, rb1, rb2, rb3, zb, ones_v, z16_v, acc_s, deg_s,
                  gs0, gs1, gs2, gs3, ss0, ss1, ss2, ss3):
    c = lax.axis_index("c")
    s = lax.axis_index("s")
    wid = c * NS + s
    rbs = [rb0, rb1, rb2, rb3]
    gss = [gs0, gs1, gs2, gs3]
    sss = [ss0, ss1, ss2, ss3]
    pltpu.sync_copy(z64_hbm, zb)
    if with_deg:
        pltpu.sync_copy(z16_hbm, z16_v)
        pltpu.sync_copy(ones_hbm, ones_v)

    def zbody(k, carry):
        pltpu.sync_copy(zb, acc_s.at[pl.ds(s * RPT + k * CH, CH)])
        if with_deg:
            pltpu.sync_copy(z16_v, deg_s.at[pl.ds(s * RPT + k * CH, CH)])
        return carry

    lax.fori_loop(0, RPT // CH, zbody, 0)
    pltpu.sync_copy(srcv_hbm.at[wid], idxs_v)
    pltpu.sync_copy(dstv_hbm.at[wid], idxd_v)
    plsc.subcore_barrier()

    rb_a, rb_b = rb0, rb1
    gs_a, gs_b = gs0, gs1
    pltpu.async_copy(y_hbm.at[idxs_v.at[0]], rb_a, gs_a)
    if with_deg:
        pltpu.async_copy(ones_v, deg_s.at[idxd_v.at[0]], ss0, add=True)
        pltpu.async_copy(ones_v, deg_s.at[idxd_v.at[1]], ss1, add=True)

    def body(t, carry):
        j0 = 2 * t
        j1 = j0 + 1
        j2 = j0 + 2
        j3 = j0 + 3
        pltpu.async_copy(y_hbm.at[idxs_v.at[j1]], rb_b, gs_b)
        pltpu.make_async_copy(y_hbm.at[idxs_v.at[j0]], rb_a, gs_a).wait()
        pltpu.sync_copy(rb_a, acc_s.at[idxd_v.at[j0]], add=True)
        if with_deg:
            pltpu.make_async_copy(ones_v, deg_s.at[idxd_v.at[0]],
                                  ss0).wait()
            pltpu.async_copy(ones_v, deg_s.at[idxd_v.at[j2]], ss0, add=True)
        pltpu.async_copy(y_hbm.at[idxs_v.at[j2]], rb_a, gs_a)
        pltpu.make_async_copy(y_hbm.at[idxs_v.at[j1]], rb_b, gs_b).wait()
        pltpu.sync_copy(rb_b, acc_s.at[idxd_v.at[j1]], add=True)
        if with_deg:
            pltpu.make_async_copy(ones_v, deg_s.at[idxd_v.at[0]],
                                  ss1).wait()
            pltpu.async_copy(ones_v, deg_s.at[idxd_v.at[j3]], ss1, add=True)
        return carry

    lax.fori_loop(0, EC // 2, body, 0)
    pltpu.make_async_copy(y_hbm.at[idxs_v.at[0]], rb_a, gs_a).wait()
    if with_deg:
        pltpu.make_async_copy(ones_v, deg_s.at[idxd_v.at[0]], ss0).wait()
        pltpu.make_async_copy(ones_v, deg_s.at[idxd_v.at[0]], ss1).wait()
    plsc.subcore_barrier()

    def wbody(k, carry):
        base = s * RPT + k * CH
        pltpu.sync_copy(acc_s.at[pl.ds(base, CH)], rb0)
        pltpu.sync_copy(rb0, acc_hbm.at[c, pl.ds(base, CH)])
        if with_deg:
            pltpu.sync_copy(deg_s.at[pl.ds(base, CH)], z16_v)
            pltpu.sync_copy(z16_v, deg_hbm.at[c, pl.ds(base, CH)])
        return carry

    lax.fori_loop(0, RPT // CH, wbody, 0)


def _gather_pairs_body(u_hbm, v_hbm, sidx_hbm, didx_hbm, gu_hbm, gv_hbm,
                       idxs_v, idxd_v, bu_a, bv_a, bu_b, bv_b,
                       su_a, sv_a, su_b, sv_b):
    c = lax.axis_index("c")
    s = lax.axis_index("s")
    wid = c * NS + s
    pltpu.sync_copy(sidx_hbm.at[wid], idxs_v)
    pltpu.sync_copy(didx_hbm.at[wid], idxd_v)
    pltpu.async_copy(u_hbm.at[idxs_v.at[0]], bu_a, su_a)
    pltpu.async_copy(v_hbm.at[idxd_v.at[0]], bv_a, sv_a)

    def body(t, carry):
        j0 = 2 * t
        j1 = j0 + 1
        j2 = (j0 + 2) % PC
        pltpu.async_copy(u_hbm.at[idxs_v.at[j1]], bu_b, su_b)
        pltpu.async_copy(v_hbm.at[idxd_v.at[j1]], bv_b, sv_b)
        pltpu.make_async_copy(u_hbm.at[idxs_v.at[j0]], bu_a, su_a).wait()
        pltpu.make_async_copy(v_hbm.at[idxd_v.at[j0]], bv_a, sv_a).wait()
        base0 = (wid * PC + j0) * CH
        pltpu.sync_copy(bu_a, gu_hbm.at[pl.ds(base0, CH)])
        pltpu.sync_copy(bv_a, gv_hbm.at[pl.ds(base0, CH)])
        pltpu.async_copy(u_hbm.at[idxs_v.at[j2]], bu_a, su_a)
        pltpu.async_copy(v_hbm.at[idxd_v.at[j2]], bv_a, sv_a)
        pltpu.make_async_copy(u_hbm.at[idxs_v.at[j1]], bu_b, su_b).wait()
        pltpu.make_async_copy(v_hbm.at[idxd_v.at[j1]], bv_b, sv_b).wait()
        base1 = (wid * PC + j1) * CH
        pltpu.sync_copy(bu_b, gu_hbm.at[pl.ds(base1, CH)])
        pltpu.sync_copy(bv_b, gv_hbm.at[pl.ds(base1, CH)])
        return carry

    lax.fori_loop(0, PC // 2, body, 0)
    pltpu.make_async_copy(u_hbm.at[idxs_v.at[0]], bu_a, su_a).wait()
    pltpu.make_async_copy(v_hbm.at[idxd_v.at[0]], bv_a, sv_a).wait()


_SC_MESH = plsc.VectorSubcoreMesh(core_axis_name="c", subcore_axis_name="s",
                                  num_cores=NC, num_subcores=NS)

_SEG_SCRATCH = ([pltpu.VMEM((EC + 2, CH), jnp.int32),
                 pltpu.VMEM((EC + 2, CH), jnp.int32)]
                + [pltpu.VMEM((CH, D), jnp.float32)] * 5
                + [pltpu.VMEM((CH, 16), jnp.float32),
                   pltpu.VMEM((CH, 16), jnp.float32),
                   pltpu.VMEM_SHARED((NP, D), jnp.float32),
                   pltpu.VMEM_SHARED((NP, 16), jnp.float32)]
                + [pltpu.SemaphoreType.DMA] * 8)

_seg_sum_deg = functools.partial(
    pl.kernel, functools.partial(_seg_sum_body, True),
    out_type=(jax.ShapeDtypeStruct((NC, NP, D), jnp.float32),
              jax.ShapeDtypeStruct((NC, NP, 16), jnp.float32)),
    mesh=_SC_MESH,
    compiler_params=pltpu.CompilerParams(use_tc_tiling_on_sc=False),
    scratch_types=_SEG_SCRATCH,
)()

_seg_sum_nodeg = functools.partial(
    pl.kernel, functools.partial(_seg_sum_body, False),
    out_type=(jax.ShapeDtypeStruct((NC, NP, D), jnp.float32),
              jax.ShapeDtypeStruct((NC, NP, 16), jnp.float32)),
    mesh=_SC_MESH,
    compiler_params=pltpu.CompilerParams(use_tc_tiling_on_sc=False),
    scratch_types=_SEG_SCRATCH,
)()

_gather_pairs = functools.partial(
    pl.kernel, _gather_pairs_body,
    out_type=(jax.ShapeDtypeStruct((PP, D), jnp.float32),
              jax.ShapeDtypeStruct((PP, D), jnp.float32)),
    mesh=_SC_MESH,
    compiler_params=pltpu.CompilerParams(use_tc_tiling_on_sc=False),
    scratch_types=[pltpu.VMEM((PC, CH), jnp.int32),
                   pltpu.VMEM((PC, CH), jnp.int32),
                   pltpu.VMEM((CH, D), jnp.float32),
                   pltpu.VMEM((CH, D), jnp.float32),
                   pltpu.VMEM((CH, D), jnp.float32),
                   pltpu.VMEM((CH, D), jnp.float32),
                   pltpu.SemaphoreType.DMA,
                   pltpu.SemaphoreType.DMA,
                   pltpu.SemaphoreType.DMA,
                   pltpu.SemaphoreType.DMA],
)()


def kernel(x, edge_index, edge_label_index, W1_l, b1_l, W1_r,
           W2_l, b2_l, W2_r, Wp1, bp1, Wp2, bp2):
    f32 = jnp.float32
    ei = edge_index.astype(jnp.int32)
    epad_s = jnp.asarray(np.arange(EP - E, dtype=np.int32) % N)
    epad_d = jnp.asarray(N + np.arange(EP - E, dtype=np.int32) % (NP - N))
    jnk = np.arange(NW * 2 * CH, dtype=np.int32).reshape(NW, 2, CH)
    jnk_s = jnp.asarray(jnk % N)
    jnk_d = jnp.asarray(N + jnk % (NP - N))
    srcv = jnp.concatenate(
        [jnp.concatenate([ei[0], epad_s]).reshape(NW, EC, CH), jnk_s], axis=1)
    dstv = jnp.concatenate(
        [jnp.concatenate([ei[1], epad_d]).reshape(NW, EC, CH), jnk_d], axis=1)
    eli = edge_label_index.astype(jnp.int32)
    ppad = jnp.asarray(np.arange(PP - P, dtype=np.int32) % N)
    sidx = jnp.concatenate([eli[0], ppad]).reshape(NW, PC, CH)
    didx = jnp.concatenate([eli[1], ppad]).reshape(NW, PC, CH)
    z64 = jnp.asarray(np.zeros((CH, D), np.float32))
    z16 = jnp.asarray(np.zeros((CH, 16), np.float32))
    ones16 = jnp.asarray(np.ones((CH, 16), np.float32))

    nb = pl.cdiv(N, 1024)
    row_spec = pl.BlockSpec((1024, D), lambda i: (i, 0))
    acc_spec = pl.BlockSpec((NC, 1024, D), lambda i: (0, i, 0))
    deg_spec = pl.BlockSpec((NC, 1024, 16), lambda i: (0, i, 0))
    w64_spec = pl.BlockSpec((D, D), lambda i: (0, 0))
    b_spec = pl.BlockSpec((1, D), lambda i: (0, 0))
    row_out = jax.ShapeDtypeStruct((N, D), f32)

    y1, r1 = pl.pallas_call(
        _stage_a,
        grid=(nb,),
        in_specs=[pl.BlockSpec((1024, 128), lambda i: (i, 0)),
                  pl.BlockSpec((D, 128), lambda i: (0, 0)),
                  pl.BlockSpec((D, 128), lambda i: (0, 0))],
        out_specs=[row_spec, row_spec],
        out_shape=[row_out, row_out],
    )(x, W1_l, W1_r)

    acc1, deg = _seg_sum_deg(y1, srcv, dstv, z64, z16, ones16)

    y2, p2, rec = pl.pallas_call(
        _stage_c,
        grid=(nb,),
        in_specs=[acc_spec, deg_spec, row_spec, b_spec, w64_spec, w64_spec,
                  b_spec],
        out_specs=[row_spec, row_spec, row_spec],
        out_shape=[row_out, row_out, row_out],
    )(acc1, deg, r1, b1_l.reshape(1, D), W2_l, W2_r, b2_l.reshape(1, D))

    acc2, _ = _seg_sum_nodeg(y2, srcv, dstv, z64, z16, ones16)

    u, v = pl.pallas_call(
        _stage_e,
        grid=(nb,),
        in_specs=[acc_spec, row_spec, row_spec, w64_spec, w64_spec],
        out_specs=[row_spec, row_spec],
        out_shape=[row_out, row_out],
    )(acc2, rec, p2, Wp1[:, :D], Wp1[:, D:])

    gu, gv = _gather_pairs(u, v, sidx, didx)

    PPH = PP // 2
    gu2 = gu.reshape(PPH, 2 * D)
    gv2 = gv.reshape(PPH, 2 * D)
    bp1c = jnp.concatenate([bp1, bp1]).reshape(1, 2 * D)
    wp2c = jnp.concatenate([Wp2[0], Wp2[0]]).reshape(1, 2 * D)
    GB = 4096
    pb = pl.cdiv(PPH, GB)
    pr2 = pl.pallas_call(
        _stage_g,
        grid=(pb,),
        in_specs=[pl.BlockSpec((GB, 2 * D), lambda i: (i, 0)),
                  pl.BlockSpec((GB, 2 * D), lambda i: (i, 0)),
                  pl.BlockSpec((1, 2 * D), lambda i: (0, 0)),
                  pl.BlockSpec((1, 2 * D), lambda i: (0, 0)),
                  pl.BlockSpec((1, 1), lambda i: (0, 0))],
        out_specs=pl.BlockSpec((GB, 2), lambda i: (i, 0)),
        out_shape=jax.ShapeDtypeStruct((PPH, 2), f32),
    )(gu2, gv2, bp1c, wp2c, bp2.reshape(1, 1))

    return pr2.reshape(PP)[:P]
